# baseline (speedup 1.0000x reference)
import functools

import jax
import jax.numpy as jnp
from jax import lax
from jax.experimental import pallas as pl
from jax.experimental.pallas import tpu as pltpu
from jax.experimental.pallas import tpu_sc as plsc

N = 10000
E = 320000
D = 128
ALPHA = 0.2

NC = 2
NS = 16
NW = NC * NS
CH = 128
CPT = 80
EPAD = NW * CPT * CH
NP = 10240
NPT = NP // NS
NPA = 10112
NPTA = NPA // NS

def _deg_body(dst_hbm, out_hbm, dst_v, ones_v, zeros_v, deg_sh, sem):
    c = lax.axis_index("c")
    s = lax.axis_index("s")
    wid = c * NS + s
    for i in range(CH // 16):
        ones_v[pl.ds(i * 16, 16)] = jnp.ones((16,), jnp.float32)
    for i in range(NPT // 16):
        zeros_v[pl.ds(i * 16, 16)] = jnp.zeros((16,), jnp.float32)
    pltpu.sync_copy(zeros_v, deg_sh.at[pl.ds(s * NPT, NPT)])
    pltpu.async_copy(dst_hbm.at[wid], dst_v, sem).wait()
    plsc.subcore_barrier()

    def body(j, carry):
        pltpu.sync_copy(ones_v, deg_sh.at[dst_v.at[j]], add=True)
        return carry

    lax.fori_loop(0, CPT, body, 0)
    plsc.subcore_barrier()
    pltpu.sync_copy(deg_sh.at[pl.ds(s * NPT, NPT)],
                    out_hbm.at[c, pl.ds(s * NPT, NPT)])


def _agg_body(y_hbm, idx_hbm, out_hbm,
              idxb0, idxb1, rows0, rows1, zeros_v, agg_sh,
              gsem0, gsem1, ssem0, ssem1, isem0, isem1, zsem):
    c = lax.axis_index("c")
    s = lax.axis_index("s")
    wid = c * NS + s
    for i in range(8):
        for k in range(D // 16):
            zeros_v[i, pl.ds(k * 16, 16)] = jnp.zeros((16,), jnp.float32)
    for j in range(NPTA // 8):
        pltpu.async_copy(zeros_v, agg_sh.at[pl.ds(s * NPTA + j * 8, 8)],
                         zsem)
    pltpu.sync_copy(idx_hbm.at[wid, 0], idxb0)
    for j in range(NPTA // 8):
        pltpu.make_async_copy(zeros_v,
                              agg_sh.at[pl.ds(s * NPTA + j * 8, 8)],
                              zsem).wait()
    plsc.subcore_barrier()

    pltpu.async_copy(y_hbm.at[idxb0.at[0]], rows0, gsem0)
    pltpu.async_copy(idx_hbm.at[wid, 1], idxb1, isem1)

    def body(t, carry):
        j1 = 2 * t + 1
        more = t < CPT // 2 - 1
        pltpu.make_async_copy(y_hbm.at[idxb0.at[0]], rows0, gsem0).wait()
        pltpu.make_async_copy(idx_hbm.at[wid, j1], idxb1, isem1).wait()
        pltpu.async_copy(y_hbm.at[idxb1.at[0]], rows1, gsem1)
        pltpu.async_copy(rows0, agg_sh.at[idxb0.at[1]], ssem0, add=True)
        pltpu.make_async_copy(y_hbm.at[idxb1.at[0]], rows1, gsem1).wait()
        pltpu.make_async_copy(rows0, agg_sh.at[idxb0.at[1]], ssem0).wait()

        @pl.when(more)
        def _():
            pltpu.async_copy(idx_hbm.at[wid, j1 + 1], idxb0, isem0)

        pltpu.async_copy(rows1, agg_sh.at[idxb1.at[1]], ssem1, add=True)

        @pl.when(more)
        def _():
            pltpu.make_async_copy(idx_hbm.at[wid, j1 + 1], idxb0, isem0).wait()
            pltpu.async_copy(y_hbm.at[idxb0.at[0]], rows0, gsem0)

        pltpu.make_async_copy(rows1, agg_sh.at[idxb1.at[1]], ssem1).wait()

        @pl.when(more)
        def _():
            pltpu.async_copy(idx_hbm.at[wid, j1 + 2], idxb1, isem1)

        return carry

    lax.fori_loop(0, CPT // 2, body, 0)
    plsc.subcore_barrier()

    wchunks = [(k * 128, 128) for k in range(4)] + [(512, NPTA - 512)]
    for off, ln in wchunks:
        pltpu.async_copy(agg_sh.at[pl.ds(s * NPTA + off, ln)],
                         out_hbm.at[c, pl.ds(s * NPTA + off, ln)], zsem)
    for off, ln in wchunks:
        pltpu.make_async_copy(agg_sh.at[pl.ds(s * NPTA + off, ln)],
                              out_hbm.at[c, pl.ds(s * NPTA + off, ln)],
                              zsem).wait()


@functools.lru_cache(maxsize=None)
def _sc_kernels():
    mesh = plsc.VectorSubcoreMesh(
        core_axis_name="c", subcore_axis_name="s",
        num_cores=NC, num_subcores=NS)
    deg_kernel = pl.kernel(
        _deg_body,
        out_type=jax.ShapeDtypeStruct((NC, NP), jnp.float32),
        mesh=mesh,
        scratch_types=[
            pltpu.VMEM((CPT, CH), jnp.int32),
            pltpu.VMEM((CH,), jnp.float32),
            pltpu.VMEM((NPT,), jnp.float32),
            pltpu.VMEM_SHARED((NP,), jnp.float32),
            pltpu.SemaphoreType.DMA,
        ],
    )
    agg_kernel = pl.kernel(
        _agg_body,
        out_type=jax.ShapeDtypeStruct((NC, NP, D), jnp.float32),
        mesh=mesh,
        scratch_types=[
            pltpu.VMEM((2, CH), jnp.int32),
            pltpu.VMEM((2, CH), jnp.int32),
            pltpu.VMEM((CH, D), jnp.float32),
            pltpu.VMEM((CH, D), jnp.float32),
            pltpu.VMEM((8, D), jnp.float32),
            pltpu.VMEM_SHARED((NPA, D), jnp.float32),
            pltpu.SemaphoreType.DMA,
            pltpu.SemaphoreType.DMA,
            pltpu.SemaphoreType.DMA,
            pltpu.SemaphoreType.DMA,
            pltpu.SemaphoreType.DMA,
            pltpu.SemaphoreType.DMA,
            pltpu.SemaphoreType.DMA,
        ],
    )
    return deg_kernel, agg_kernel


def _prep_body(degp_ref, x_ref, dinv_ref, y_ref):
    d = degp_ref[0, :] + degp_ref[1, :] + 1.0
    dinv = lax.rsqrt(d).reshape(-1, 1)
    dinv_ref[...] = dinv
    y_ref[...] = x_ref[...] * dinv


def _prep(degp, xp):
    blk = 128
    grid = NP // blk
    return pl.pallas_call(
        _prep_body,
        grid=(grid,),
        in_specs=[
            pl.BlockSpec((NC, blk), lambda i: (0, i)),
            pl.BlockSpec((blk, D), lambda i: (i, 0)),
        ],
        out_specs=[
            pl.BlockSpec((blk, 1), lambda i: (i, 0)),
            pl.BlockSpec((blk, D), lambda i: (i, 0)),
        ],
        out_shape=[
            jax.ShapeDtypeStruct((NP, 1), jnp.float32),
            jax.ShapeDtypeStruct((NP, D), jnp.float32),
        ],
    )(degp, xp)


def _final_body(p_ref, dinv_ref, y_ref, x0_ref, x_ref, w_ref, o_ref):
    ssum = p_ref[0] + p_ref[1] + y_ref[...]
    agg = ssum * dinv_ref[...]
    h = (1.0 - ALPHA) * agg + ALPHA * x0_ref[...]
    mm = jnp.dot(h, w_ref[...], preferred_element_type=jnp.float32)
    o_ref[...] = x_ref[...] + jnp.maximum(mm, 0.0)


def _final(parts, dinv, y, x0p, xp, W1):
    blk = 512
    grid = NP // blk
    return pl.pallas_call(
        _final_body,
        grid=(grid,),
        in_specs=[
            pl.BlockSpec((NC, blk, D), lambda i: (0, i, 0)),
            pl.BlockSpec((blk, 1), lambda i: (i, 0)),
            pl.BlockSpec((blk, D), lambda i: (i, 0)),
            pl.BlockSpec((blk, D), lambda i: (i, 0)),
            pl.BlockSpec((blk, D), lambda i: (i, 0)),
            pl.BlockSpec((D, D), lambda i: (0, 0)),
        ],
        out_specs=pl.BlockSpec((blk, D), lambda i: (i, 0)),
        out_shape=jax.ShapeDtypeStruct((NP, D), jnp.float32),
    )(parts, dinv, y, x0p, xp, W1)


def kernel(x, x0, edge_index, W1):
    src = edge_index[0]
    dst = edge_index[1]
    pad = jnp.full((EPAD - E,), N, jnp.int32)
    srcp = jnp.concatenate([src, pad]).reshape(NW, CPT, CH)
    dstp = jnp.concatenate([dst, pad]).reshape(NW, CPT, CH)
    idxp = jnp.stack([srcp, dstp], axis=2)
    xp = jnp.pad(x, ((0, NP - N), (0, 0)))
    x0p = jnp.pad(x0, ((0, NP - N), (0, 0)))

    deg_kernel, agg_kernel = _sc_kernels()
    degp = deg_kernel(dstp)
    dinv, y = _prep(degp, xp)
    parts = agg_kernel(y, idxp)
    outp = _final(parts, dinv, y, x0p, xp, W1)
    return outp[:N]


# speedup vs baseline: 1.0670x; 1.0670x over previous
import functools

import jax
import jax.numpy as jnp
from jax import lax
from jax.experimental import pallas as pl
from jax.experimental.pallas import tpu as pltpu
from jax.experimental.pallas import tpu_sc as plsc

N = 10000
E = 320000
D = 128
ALPHA = 0.2

NC = 2
NS = 16
NW = NC * NS
CH = 128
CPT = 80
EPAD = NW * CPT * CH
NP = 10240
NPT = NP // NS
NPA = 10112
NPTA = NPA // NS

def _deg_body(dst_hbm, out_hbm, dst_v, ones_v, zeros_v, deg_sh, sem):
    c = lax.axis_index("c")
    s = lax.axis_index("s")
    wid = c * NS + s
    for i in range(CH // 16):
        ones_v[pl.ds(i * 16, 16)] = jnp.ones((16,), jnp.float32)
    for i in range(NPT // 16):
        zeros_v[pl.ds(i * 16, 16)] = jnp.zeros((16,), jnp.float32)
    pltpu.sync_copy(zeros_v, deg_sh.at[pl.ds(s * NPT, NPT)])
    pltpu.async_copy(dst_hbm.at[wid], dst_v, sem).wait()
    plsc.subcore_barrier()

    def body(j, carry):
        pltpu.sync_copy(ones_v, deg_sh.at[dst_v.at[j]], add=True)
        return carry

    lax.fori_loop(0, CPT, body, 0)
    plsc.subcore_barrier()
    pltpu.sync_copy(deg_sh.at[pl.ds(s * NPT, NPT)],
                    out_hbm.at[c, pl.ds(s * NPT, NPT)])


def _agg_body(y_hbm, idx_hbm, out_hbm,
              idxga, idxgb, rows0, rows1, zeros_v, agg_sh,
              gsem0, gsem1, ssem0, ssem1, isema, isemb, zsem):
    c = lax.axis_index("c")
    s = lax.axis_index("s")
    wid = c * NS + s
    for i in range(8):
        for k in range(D // 16):
            zeros_v[i, pl.ds(k * 16, 16)] = jnp.zeros((16,), jnp.float32)
    for j in range(NPTA // 8):
        pltpu.async_copy(zeros_v, agg_sh.at[pl.ds(s * NPTA + j * 8, 8)],
                         zsem)
    pltpu.sync_copy(idx_hbm.at[wid, 0], idxga)
    for j in range(NPTA // 8):
        pltpu.make_async_copy(zeros_v,
                              agg_sh.at[pl.ds(s * NPTA + j * 8, 8)],
                              zsem).wait()
    plsc.subcore_barrier()

    def g_start(idxg, b, rows, gsem):
        pltpu.async_copy(y_hbm.at[idxg.at[b, 0]], rows, gsem)

    def g_wait(idxg, b, rows, gsem):
        pltpu.make_async_copy(y_hbm.at[idxg.at[b, 0]], rows, gsem).wait()

    def sc_do(idxg, b, rows, ssem):
        pltpu.async_copy(rows, agg_sh.at[idxg.at[b, 1]], ssem, add=True)
        pltpu.make_async_copy(rows, agg_sh.at[idxg.at[b, 1]], ssem).wait()

    pltpu.async_copy(idx_hbm.at[wid, 1], idxgb, isemb)
    g_start(idxga, 0, rows0, gsem0)
    g_start(idxga, 1, rows1, gsem1)

    def body(u, carry):
        more = u < CPT // 4 - 1
        g_wait(idxga, 0, rows0, gsem0)
        sc_do(idxga, 0, rows0, ssem0)
        pltpu.make_async_copy(idx_hbm.at[wid, 2 * u + 1], idxgb, isemb).wait()
        g_start(idxgb, 0, rows0, gsem0)
        g_wait(idxga, 1, rows1, gsem1)
        sc_do(idxga, 1, rows1, ssem1)
        g_start(idxgb, 1, rows1, gsem1)

        @pl.when(more)
        def _():
            pltpu.async_copy(idx_hbm.at[wid, 2 * u + 2], idxga, isema)

        g_wait(idxgb, 0, rows0, gsem0)
        sc_do(idxgb, 0, rows0, ssem0)

        @pl.when(more)
        def _():
            pltpu.make_async_copy(idx_hbm.at[wid, 2 * u + 2], idxga,
                                  isema).wait()
            g_start(idxga, 0, rows0, gsem0)

        g_wait(idxgb, 1, rows1, gsem1)
        sc_do(idxgb, 1, rows1, ssem1)

        @pl.when(more)
        def _():
            g_start(idxga, 1, rows1, gsem1)
            pltpu.async_copy(idx_hbm.at[wid, 2 * u + 3], idxgb, isemb)

        return carry

    lax.fori_loop(0, CPT // 4, body, 0)
    plsc.subcore_barrier()

    wchunks = [(k * 128, 128) for k in range(4)] + [(512, NPTA - 512)]
    for off, ln in wchunks:
        pltpu.async_copy(agg_sh.at[pl.ds(s * NPTA + off, ln)],
                         out_hbm.at[c, pl.ds(s * NPTA + off, ln)], zsem)
    for off, ln in wchunks:
        pltpu.make_async_copy(agg_sh.at[pl.ds(s * NPTA + off, ln)],
                              out_hbm.at[c, pl.ds(s * NPTA + off, ln)],
                              zsem).wait()


@functools.lru_cache(maxsize=None)
def _sc_kernels():
    mesh = plsc.VectorSubcoreMesh(
        core_axis_name="c", subcore_axis_name="s",
        num_cores=NC, num_subcores=NS)
    deg_kernel = pl.kernel(
        _deg_body,
        out_type=jax.ShapeDtypeStruct((NC, NP), jnp.float32),
        mesh=mesh,
        scratch_types=[
            pltpu.VMEM((CPT, CH), jnp.int32),
            pltpu.VMEM((CH,), jnp.float32),
            pltpu.VMEM((NPT,), jnp.float32),
            pltpu.VMEM_SHARED((NP,), jnp.float32),
            pltpu.SemaphoreType.DMA,
        ],
    )
    agg_kernel = pl.kernel(
        _agg_body,
        out_type=jax.ShapeDtypeStruct((NC, NP, D), jnp.float32),
        mesh=mesh,
        scratch_types=[
            pltpu.VMEM((2, 2, CH), jnp.int32),
            pltpu.VMEM((2, 2, CH), jnp.int32),
            pltpu.VMEM((CH, D), jnp.float32),
            pltpu.VMEM((CH, D), jnp.float32),
            pltpu.VMEM((8, D), jnp.float32),
            pltpu.VMEM_SHARED((NPA, D), jnp.float32),
            pltpu.SemaphoreType.DMA,
            pltpu.SemaphoreType.DMA,
            pltpu.SemaphoreType.DMA,
            pltpu.SemaphoreType.DMA,
            pltpu.SemaphoreType.DMA,
            pltpu.SemaphoreType.DMA,
            pltpu.SemaphoreType.DMA,
        ],
    )
    return deg_kernel, agg_kernel


def _prep_body(degp_ref, x_ref, dinv_ref, y_ref):
    d = degp_ref[0, :] + degp_ref[1, :] + 1.0
    dinv = lax.rsqrt(d).reshape(-1, 1)
    dinv_ref[...] = dinv
    y_ref[...] = x_ref[...] * dinv


def _prep(degp, xp):
    blk = 128
    grid = NP // blk
    return pl.pallas_call(
        _prep_body,
        grid=(grid,),
        in_specs=[
            pl.BlockSpec((NC, blk), lambda i: (0, i)),
            pl.BlockSpec((blk, D), lambda i: (i, 0)),
        ],
        out_specs=[
            pl.BlockSpec((blk, 1), lambda i: (i, 0)),
            pl.BlockSpec((blk, D), lambda i: (i, 0)),
        ],
        out_shape=[
            jax.ShapeDtypeStruct((NP, 1), jnp.float32),
            jax.ShapeDtypeStruct((NP, D), jnp.float32),
        ],
    )(degp, xp)


def _final_body(p_ref, dinv_ref, y_ref, x0_ref, x_ref, w_ref, o_ref):
    ssum = p_ref[0] + p_ref[1] + y_ref[...]
    agg = ssum * dinv_ref[...]
    h = (1.0 - ALPHA) * agg + ALPHA * x0_ref[...]
    mm = jnp.dot(h, w_ref[...], preferred_element_type=jnp.float32)
    o_ref[...] = x_ref[...] + jnp.maximum(mm, 0.0)


def _final(parts, dinv, y, x0p, xp, W1):
    blk = 512
    grid = NP // blk
    return pl.pallas_call(
        _final_body,
        grid=(grid,),
        in_specs=[
            pl.BlockSpec((NC, blk, D), lambda i: (0, i, 0)),
            pl.BlockSpec((blk, 1), lambda i: (i, 0)),
            pl.BlockSpec((blk, D), lambda i: (i, 0)),
            pl.BlockSpec((blk, D), lambda i: (i, 0)),
            pl.BlockSpec((blk, D), lambda i: (i, 0)),
            pl.BlockSpec((D, D), lambda i: (0, 0)),
        ],
        out_specs=pl.BlockSpec((blk, D), lambda i: (i, 0)),
        out_shape=jax.ShapeDtypeStruct((NP, D), jnp.float32),
    )(parts, dinv, y, x0p, xp, W1)


def kernel(x, x0, edge_index, W1):
    src = edge_index[0]
    dst = edge_index[1]
    pad = jnp.full((EPAD - E,), N, jnp.int32)
    srcp = jnp.concatenate([src, pad]).reshape(NW, CPT, CH)
    dstp = jnp.concatenate([dst, pad]).reshape(NW, CPT, CH)
    idxp = jnp.stack([srcp, dstp], axis=2)
    idxp = idxp.reshape(NW, CPT // 2, 2, 2, CH)
    xp = jnp.pad(x, ((0, NP - N), (0, 0)))
    x0p = jnp.pad(x0, ((0, NP - N), (0, 0)))

    deg_kernel, agg_kernel = _sc_kernels()
    degp = deg_kernel(dstp)
    dinv, y = _prep(degp, xp)
    parts = agg_kernel(y, idxp)
    outp = _final(parts, dinv, y, x0p, xp, W1)
    return outp[:N]


# speedup vs baseline: 2.6474x; 2.4812x over previous
import functools

import jax
import jax.numpy as jnp
from jax import lax
from jax.experimental import pallas as pl
from jax.experimental.pallas import tpu as pltpu
from jax.experimental.pallas import tpu_sc as plsc

N = 10000
E = 320000
D = 128
ALPHA = 0.2

NC = 2
NS = 16
NW = NC * NS
CH = 128
CPT = 80
EPAD = NW * CPT * CH
NP = 10240
NPT = NP // NS
NPA = 10112
NPTA = NPA // NS

def _deg_body(dst_hbm, out_hbm, dst_v, ones_v, zeros_v, deg_sh, sem):
    c = lax.axis_index("c")
    s = lax.axis_index("s")
    wid = c * NS + s
    for i in range(CH // 16):
        ones_v[pl.ds(i * 16, 16)] = jnp.ones((16,), jnp.float32)
    for i in range(NPT // 16):
        zeros_v[pl.ds(i * 16, 16)] = jnp.zeros((16,), jnp.float32)
    pltpu.sync_copy(zeros_v, deg_sh.at[pl.ds(s * NPT, NPT)])
    pltpu.async_copy(dst_hbm.at[wid], dst_v, sem).wait()
    plsc.subcore_barrier()

    def body(j, carry):
        pltpu.sync_copy(ones_v, deg_sh.at[dst_v.at[j]], add=True)
        return carry

    lax.fori_loop(0, CPT, body, 0)
    plsc.subcore_barrier()
    pltpu.sync_copy(deg_sh.at[pl.ds(s * NPT, NPT)],
                    out_hbm.at[c, pl.ds(s * NPT, NPT)])


def _agg_body(y_hbm, idx_hbm, out_hbm,
              idxga, idxgb, rows0, rows1, zeros_v, agg_sh,
              gsem0, gsem1, ssem0, ssem1, isema, isemb, zsem):
    c = lax.axis_index("c")
    s = lax.axis_index("s")
    wid = c * NS + s
    for i in range(8):
        for k in range(D // 16):
            zeros_v[i, pl.ds(k * 16, 16)] = jnp.zeros((16,), jnp.float32)
    for j in range(NPTA // 8):
        pltpu.async_copy(zeros_v, agg_sh.at[pl.ds(s * NPTA + j * 8, 8)],
                         zsem)
    pltpu.sync_copy(idx_hbm.at[wid, 0], idxga)
    for j in range(NPTA // 8):
        pltpu.make_async_copy(zeros_v,
                              agg_sh.at[pl.ds(s * NPTA + j * 8, 8)],
                              zsem).wait()
    plsc.subcore_barrier()

    def g_start(idxg, b, rows, gsem):
        pltpu.async_copy(y_hbm.at[idxg.at[b, 0]], rows, gsem)

    def g_wait(idxg, b, rows, gsem):
        pltpu.make_async_copy(y_hbm.at[idxg.at[b, 0]], rows, gsem).wait()

    def sc_do(idxg, b, rows, ssem):
        pltpu.async_copy(rows, agg_sh.at[idxg.at[b, 1]], ssem, add=True)
        pltpu.make_async_copy(rows, agg_sh.at[idxg.at[b, 1]], ssem).wait()

    pltpu.async_copy(idx_hbm.at[wid, 1], idxgb, isemb)
    g_start(idxga, 0, rows0, gsem0)
    g_start(idxga, 1, rows1, gsem1)

    def body(u, carry):
        more = u < CPT // 4 - 1
        g_wait(idxga, 0, rows0, gsem0)
        sc_do(idxga, 0, rows0, ssem0)
        pltpu.make_async_copy(idx_hbm.at[wid, 2 * u + 1], idxgb, isemb).wait()
        g_start(idxgb, 0, rows0, gsem0)
        g_wait(idxga, 1, rows1, gsem1)
        sc_do(idxga, 1, rows1, ssem1)
        g_start(idxgb, 1, rows1, gsem1)

        @pl.when(more)
        def _():
            pltpu.async_copy(idx_hbm.at[wid, 2 * u + 2], idxga, isema)

        g_wait(idxgb, 0, rows0, gsem0)
        sc_do(idxgb, 0, rows0, ssem0)

        @pl.when(more)
        def _():
            pltpu.make_async_copy(idx_hbm.at[wid, 2 * u + 2], idxga,
                                  isema).wait()
            g_start(idxga, 0, rows0, gsem0)

        g_wait(idxgb, 1, rows1, gsem1)
        sc_do(idxgb, 1, rows1, ssem1)

        @pl.when(more)
        def _():
            g_start(idxga, 1, rows1, gsem1)
            pltpu.async_copy(idx_hbm.at[wid, 2 * u + 3], idxgb, isemb)

        return carry

    lax.fori_loop(0, CPT // 4, body, 0)
    plsc.subcore_barrier()

    wchunks = [(k * 128, 128) for k in range(4)] + [(512, NPTA - 512)]
    for off, ln in wchunks:
        pltpu.async_copy(agg_sh.at[pl.ds(s * NPTA + off, ln)],
                         out_hbm.at[c, pl.ds(s * NPTA + off, ln)], zsem)
    for off, ln in wchunks:
        pltpu.make_async_copy(agg_sh.at[pl.ds(s * NPTA + off, ln)],
                              out_hbm.at[c, pl.ds(s * NPTA + off, ln)],
                              zsem).wait()


@functools.lru_cache(maxsize=None)
def _sc_kernels():
    mesh = plsc.VectorSubcoreMesh(
        core_axis_name="c", subcore_axis_name="s",
        num_cores=NC, num_subcores=NS)
    deg_kernel = pl.kernel(
        _deg_body,
        out_type=jax.ShapeDtypeStruct((NC, NP), jnp.float32),
        mesh=mesh,
        scratch_types=[
            pltpu.VMEM((CPT, CH), jnp.int32),
            pltpu.VMEM((CH,), jnp.float32),
            pltpu.VMEM((NPT,), jnp.float32),
            pltpu.VMEM_SHARED((NP,), jnp.float32),
            pltpu.SemaphoreType.DMA,
        ],
    )
    agg_kernel = pl.kernel(
        _agg_body,
        out_type=jax.ShapeDtypeStruct((NC, NP, D), jnp.float32),
        mesh=mesh,
        scratch_types=[
            pltpu.VMEM((2, 2, CH), jnp.int32),
            pltpu.VMEM((2, 2, CH), jnp.int32),
            pltpu.VMEM((CH, D), jnp.float32),
            pltpu.VMEM((CH, D), jnp.float32),
            pltpu.VMEM((8, D), jnp.float32),
            pltpu.VMEM_SHARED((NPA, D), jnp.float32),
            pltpu.SemaphoreType.DMA,
            pltpu.SemaphoreType.DMA,
            pltpu.SemaphoreType.DMA,
            pltpu.SemaphoreType.DMA,
            pltpu.SemaphoreType.DMA,
            pltpu.SemaphoreType.DMA,
            pltpu.SemaphoreType.DMA,
        ],
    )
    return deg_kernel, agg_kernel


def _prep_body(degp_ref, x_ref, dinv_ref, y_ref):
    d = degp_ref[0, :] + degp_ref[1, :] + 1.0
    dinv = lax.rsqrt(d).reshape(-1, 1)
    dinv_ref[...] = dinv
    y_ref[...] = x_ref[...] * dinv


def _prep(degp, xp):
    blk = 128
    grid = NP // blk
    return pl.pallas_call(
        _prep_body,
        grid=(grid,),
        in_specs=[
            pl.BlockSpec((NC, blk), lambda i: (0, i)),
            pl.BlockSpec((blk, D), lambda i: (i, 0)),
        ],
        out_specs=[
            pl.BlockSpec((blk, 1), lambda i: (i, 0)),
            pl.BlockSpec((blk, D), lambda i: (i, 0)),
        ],
        out_shape=[
            jax.ShapeDtypeStruct((NP, 1), jnp.float32),
            jax.ShapeDtypeStruct((NP, D), jnp.float32),
        ],
    )(degp, xp)


def _final_body(p_ref, dinv_ref, y_ref, x0_ref, x_ref, w_ref, o_ref):
    ssum = p_ref[0] + p_ref[1] + y_ref[...]
    agg = ssum * dinv_ref[...]
    h = (1.0 - ALPHA) * agg + ALPHA * x0_ref[...]
    mm = jnp.dot(h, w_ref[...], preferred_element_type=jnp.float32)
    o_ref[...] = x_ref[...] + jnp.maximum(mm, 0.0)


def _final(parts, dinv, y, x0p, xp, W1):
    blk = 512
    grid = NP // blk
    return pl.pallas_call(
        _final_body,
        grid=(grid,),
        in_specs=[
            pl.BlockSpec((NC, blk, D), lambda i: (0, i, 0)),
            pl.BlockSpec((blk, 1), lambda i: (i, 0)),
            pl.BlockSpec((blk, D), lambda i: (i, 0)),
            pl.BlockSpec((blk, D), lambda i: (i, 0)),
            pl.BlockSpec((blk, D), lambda i: (i, 0)),
            pl.BlockSpec((D, D), lambda i: (0, 0)),
        ],
        out_specs=pl.BlockSpec((blk, D), lambda i: (i, 0)),
        out_shape=jax.ShapeDtypeStruct((NP, D), jnp.float32),
    )(parts, dinv, y, x0p, xp, W1)


def kernel(x, x0, edge_index, W1):
    src = edge_index[0]
    dst = edge_index[1]
    pidx = jnp.arange(EPAD - E, dtype=jnp.int32)
    pad_src = N + pidx % (NP - N)
    pad_dst = N + pidx % (NPA - N)
    srcp = jnp.concatenate([src, pad_src]).reshape(NW, CPT, CH)
    dstp = jnp.concatenate([dst, pad_dst]).reshape(NW, CPT, CH)
    idxp = jnp.stack([srcp, dstp], axis=2)
    idxp = idxp.reshape(NW, CPT // 2, 2, 2, CH)
    xp = jnp.pad(x, ((0, NP - N), (0, 0)))
    x0p = jnp.pad(x0, ((0, NP - N), (0, 0)))

    deg_kernel, agg_kernel = _sc_kernels()
    degp = deg_kernel(dstp)
    dinv, y = _prep(degp, xp)
    parts = agg_kernel(y, idxp)
    outp = _final(parts, dinv, y, x0p, xp, W1)
    return outp[:N]


# speedup vs baseline: 3.2555x; 1.2297x over previous
import functools

import jax
import jax.numpy as jnp
import numpy as np
from jax import lax
from jax.experimental import pallas as pl
from jax.experimental.pallas import tpu as pltpu
from jax.experimental.pallas import tpu_sc as plsc

N = 10000
E = 320000
D = 128
ALPHA = 0.2

NC = 2
NS = 16
NW = NC * NS
CH = 128
CPT = 80
EPAD = NW * CPT * CH
NP = 10240
NPT = NP // NS
NPA = 10112
NPTA = NPA // NS

_PIDX = np.arange(EPAD - E)
_PAD_SRC = jnp.asarray((_PIDX % N).astype(np.int32))
_PAD_DST = jnp.asarray((N + _PIDX % (NPA - N)).astype(np.int32))

def _deg_body(dst_hbm, out_hbm, dst_v, ones_v, zeros_v, deg_sh, sem):
    c = lax.axis_index("c")
    s = lax.axis_index("s")
    wid = c * NS + s
    for i in range(CH // 16):
        ones_v[pl.ds(i * 16, 16)] = jnp.ones((16,), jnp.float32)
    for i in range(NPT // 16):
        zeros_v[pl.ds(i * 16, 16)] = jnp.zeros((16,), jnp.float32)
    pltpu.sync_copy(zeros_v, deg_sh.at[pl.ds(s * NPT, NPT)])
    pltpu.async_copy(dst_hbm.at[wid], dst_v, sem).wait()
    plsc.subcore_barrier()

    def body(j, carry):
        pltpu.sync_copy(ones_v, deg_sh.at[dst_v.at[j, 0]], add=True)
        pltpu.sync_copy(ones_v, deg_sh.at[dst_v.at[j, 1]], add=True)
        return carry

    lax.fori_loop(0, CPT // 2, body, 0)
    plsc.subcore_barrier()
    pltpu.sync_copy(deg_sh.at[pl.ds(s * NPT, NPT)],
                    out_hbm.at[c, pl.ds(s * NPT, NPT)])


def _agg_body(y_hbm, src_hbm, dst_hbm, out_hbm,
              srcga, srcgb, dstga, dstgb, rows0, rows1, zeros_v, agg_sh,
              gsem0, gsem1, ssem0, ssem1, isema, isemb, zsem):
    c = lax.axis_index("c")
    s = lax.axis_index("s")
    wid = c * NS + s
    for i in range(8):
        for k in range(D // 16):
            zeros_v[i, pl.ds(k * 16, 16)] = jnp.zeros((16,), jnp.float32)
    for j in range(NPTA // 8):
        pltpu.async_copy(zeros_v, agg_sh.at[pl.ds(s * NPTA + j * 8, 8)],
                         zsem)
    pltpu.sync_copy(src_hbm.at[wid, 0], srcga)
    pltpu.sync_copy(dst_hbm.at[wid, 0], dstga)
    for j in range(NPTA // 8):
        pltpu.make_async_copy(zeros_v,
                              agg_sh.at[pl.ds(s * NPTA + j * 8, 8)],
                              zsem).wait()
    plsc.subcore_barrier()

    def i_start(pair, srcg, dstg, isem):
        pltpu.async_copy(src_hbm.at[wid, pair], srcg, isem)
        pltpu.async_copy(dst_hbm.at[wid, pair], dstg, isem)

    def i_wait(pair, srcg, dstg, isem):
        pltpu.make_async_copy(src_hbm.at[wid, pair], srcg, isem).wait()
        pltpu.make_async_copy(dst_hbm.at[wid, pair], dstg, isem).wait()

    def g_start(srcg, b, rows, gsem):
        pltpu.async_copy(y_hbm.at[srcg.at[b]], rows, gsem)

    def g_wait(srcg, b, rows, gsem):
        pltpu.make_async_copy(y_hbm.at[srcg.at[b]], rows, gsem).wait()

    def sc_do(dstg, b, rows, ssem):
        pltpu.async_copy(rows, agg_sh.at[dstg.at[b]], ssem, add=True)
        pltpu.make_async_copy(rows, agg_sh.at[dstg.at[b]], ssem).wait()

    i_start(1, srcgb, dstgb, isemb)
    g_start(srcga, 0, rows0, gsem0)
    g_start(srcga, 1, rows1, gsem1)

    def body(u, carry):
        more = u < CPT // 4 - 1
        g_wait(srcga, 0, rows0, gsem0)
        sc_do(dstga, 0, rows0, ssem0)
        i_wait(2 * u + 1, srcgb, dstgb, isemb)
        g_start(srcgb, 0, rows0, gsem0)
        g_wait(srcga, 1, rows1, gsem1)
        sc_do(dstga, 1, rows1, ssem1)
        g_start(srcgb, 1, rows1, gsem1)

        @pl.when(more)
        def _():
            i_start(2 * u + 2, srcga, dstga, isema)

        g_wait(srcgb, 0, rows0, gsem0)
        sc_do(dstgb, 0, rows0, ssem0)

        @pl.when(more)
        def _():
            i_wait(2 * u + 2, srcga, dstga, isema)
            g_start(srcga, 0, rows0, gsem0)

        g_wait(srcgb, 1, rows1, gsem1)
        sc_do(dstgb, 1, rows1, ssem1)

        @pl.when(more)
        def _():
            g_start(srcga, 1, rows1, gsem1)
            i_start(2 * u + 3, srcgb, dstgb, isemb)

        return carry

    lax.fori_loop(0, CPT // 4, body, 0)
    plsc.subcore_barrier()

    wchunks = [(k * 128, 128) for k in range(4)] + [(512, NPTA - 512)]
    for off, ln in wchunks:
        pltpu.async_copy(agg_sh.at[pl.ds(s * NPTA + off, ln)],
                         out_hbm.at[c, pl.ds(s * NPTA + off, ln)], zsem)
    for off, ln in wchunks:
        pltpu.make_async_copy(agg_sh.at[pl.ds(s * NPTA + off, ln)],
                              out_hbm.at[c, pl.ds(s * NPTA + off, ln)],
                              zsem).wait()


@functools.lru_cache(maxsize=None)
def _sc_kernels():
    mesh = plsc.VectorSubcoreMesh(
        core_axis_name="c", subcore_axis_name="s",
        num_cores=NC, num_subcores=NS)
    deg_kernel = pl.kernel(
        _deg_body,
        out_type=jax.ShapeDtypeStruct((NC, NP), jnp.float32),
        mesh=mesh,
        scratch_types=[
            pltpu.VMEM((CPT // 2, 2, CH), jnp.int32),
            pltpu.VMEM((CH,), jnp.float32),
            pltpu.VMEM((NPT,), jnp.float32),
            pltpu.VMEM_SHARED((NP,), jnp.float32),
            pltpu.SemaphoreType.DMA,
        ],
    )
    agg_kernel = pl.kernel(
        _agg_body,
        out_type=jax.ShapeDtypeStruct((NC, NP, D), jnp.float32),
        mesh=mesh,
        scratch_types=[
            pltpu.VMEM((2, CH), jnp.int32),
            pltpu.VMEM((2, CH), jnp.int32),
            pltpu.VMEM((2, CH), jnp.int32),
            pltpu.VMEM((2, CH), jnp.int32),
            pltpu.VMEM((CH, D), jnp.float32),
            pltpu.VMEM((CH, D), jnp.float32),
            pltpu.VMEM((8, D), jnp.float32),
            pltpu.VMEM_SHARED((NPA, D), jnp.float32),
            pltpu.SemaphoreType.DMA,
            pltpu.SemaphoreType.DMA,
            pltpu.SemaphoreType.DMA,
            pltpu.SemaphoreType.DMA,
            pltpu.SemaphoreType.DMA,
            pltpu.SemaphoreType.DMA,
            pltpu.SemaphoreType.DMA,
        ],
    )
    return deg_kernel, agg_kernel


def _prep_body(degt_ref, x_ref, dinv_ref, y_ref):
    dinv = lax.rsqrt(degt_ref[:, 0:1] + degt_ref[:, 1:2] + 1.0)
    dinv_ref[...] = dinv
    y_ref[...] = x_ref[...] * dinv


def _prep(degt, x):
    blk = 1000
    grid = N // blk
    return pl.pallas_call(
        _prep_body,
        grid=(grid,),
        in_specs=[
            pl.BlockSpec((blk, NC), lambda i: (i, 0)),
            pl.BlockSpec((blk, D), lambda i: (i, 0)),
        ],
        out_specs=[
            pl.BlockSpec((blk, 1), lambda i: (i, 0)),
            pl.BlockSpec((blk, D), lambda i: (i, 0)),
        ],
        out_shape=[
            jax.ShapeDtypeStruct((N, 1), jnp.float32),
            jax.ShapeDtypeStruct((N, D), jnp.float32),
        ],
    )(degt, x)


def _final_body(p_ref, dinv_ref, x0_ref, x_ref, w_ref, o_ref):
    dv = dinv_ref[...]
    xv = x_ref[...]
    agg = (p_ref[0] + p_ref[1]) * dv + xv * (dv * dv)
    h = (1.0 - ALPHA) * agg + ALPHA * x0_ref[...]
    mm = jnp.dot(h, w_ref[...], preferred_element_type=jnp.float32)
    o_ref[...] = xv + jnp.maximum(mm, 0.0)


def _final(parts, dinv, x0, x, W1):
    blk = 1000
    grid = N // blk
    return pl.pallas_call(
        _final_body,
        grid=(grid,),
        in_specs=[
            pl.BlockSpec((NC, blk, D), lambda i: (0, i, 0)),
            pl.BlockSpec((blk, 1), lambda i: (i, 0)),
            pl.BlockSpec((blk, D), lambda i: (i, 0)),
            pl.BlockSpec((blk, D), lambda i: (i, 0)),
            pl.BlockSpec((D, D), lambda i: (0, 0)),
        ],
        out_specs=pl.BlockSpec((blk, D), lambda i: (i, 0)),
        out_shape=jax.ShapeDtypeStruct((N, D), jnp.float32),
    )(parts, dinv, x0, x, W1)


def kernel(x, x0, edge_index, W1):
    src = edge_index[0]
    dst = edge_index[1]
    srcp = jnp.concatenate([src, _PAD_SRC]).reshape(NW, CPT // 2, 2, CH)
    dstp = jnp.concatenate([dst, _PAD_DST]).reshape(NW, CPT // 2, 2, CH)

    deg_kernel, agg_kernel = _sc_kernels()
    degp = deg_kernel(dstp)
    dinv, y = _prep(degp.T, x)
    parts = agg_kernel(y, srcp, dstp)
    return _final(parts, dinv, x0, x, W1)


# speedup vs baseline: 3.4114x; 1.0479x over previous
import functools

import jax
import jax.numpy as jnp
from jax import lax
from jax.experimental import pallas as pl
from jax.experimental.pallas import tpu as pltpu
from jax.experimental.pallas import tpu_sc as plsc

N = 10000
E = 320000
D = 128
ALPHA = 0.2

NC = 2
NS = 16
NW = NC * NS
CH = 125
G = 8
NG = 10
NP = 10240
NPT = NP // NS
NPA = 10112
NPTA = NPA // NS


def _deg_body(ei_hbm, out_hbm, dst_v, ones_v, zeros_v, deg_sh, sem):
    c = lax.axis_index("c")
    s = lax.axis_index("s")
    wid = c * NS + s
    for i in range(128 // 16):
        ones_v[pl.ds(i * 16, 16)] = jnp.ones((16,), jnp.float32)
    for i in range(NPT // 16):
        zeros_v[pl.ds(i * 16, 16)] = jnp.zeros((16,), jnp.float32)
    pltpu.sync_copy(zeros_v, deg_sh.at[pl.ds(s * NPT, NPT)])
    pltpu.async_copy(ei_hbm.at[1, wid], dst_v, sem).wait()
    plsc.subcore_barrier()

    def body(j, carry):
        for k in range(G):
            pltpu.sync_copy(ones_v.at[pl.ds(0, CH)],
                            deg_sh.at[dst_v.at[j, k]], add=True)
        return carry

    lax.fori_loop(0, NG, body, 0)
    plsc.subcore_barrier()
    pltpu.sync_copy(deg_sh.at[pl.ds(s * NPT, NPT)],
                    out_hbm.at[c, pl.ds(s * NPT, NPT)])


def _agg_body(y_hbm, ei_hbm, out_hbm,
              srcga, srcgb, dstga, dstgb, rows0, rows1, zeros_v, agg_sh,
              gsem0, gsem1, ssem0, ssem1, isema, isemb, zsem):
    c = lax.axis_index("c")
    s = lax.axis_index("s")
    wid = c * NS + s
    for i in range(8):
        for k in range(D // 16):
            zeros_v[i, pl.ds(k * 16, 16)] = jnp.zeros((16,), jnp.float32)
    for j in range(NPTA // 8):
        pltpu.async_copy(zeros_v, agg_sh.at[pl.ds(s * NPTA + j * 8, 8)],
                         zsem)
    pltpu.sync_copy(ei_hbm.at[0, wid, 0], srcga)
    pltpu.sync_copy(ei_hbm.at[1, wid, 0], dstga)
    for j in range(NPTA // 8):
        pltpu.make_async_copy(zeros_v,
                              agg_sh.at[pl.ds(s * NPTA + j * 8, 8)],
                              zsem).wait()
    plsc.subcore_barrier()

    def i_start(grp, srcg, dstg, isem):
        pltpu.async_copy(ei_hbm.at[0, wid, grp], srcg, isem)
        pltpu.async_copy(ei_hbm.at[1, wid, grp], dstg, isem)

    def i_wait(grp, srcg, dstg, isem):
        pltpu.make_async_copy(ei_hbm.at[0, wid, grp], srcg, isem).wait()
        pltpu.make_async_copy(ei_hbm.at[1, wid, grp], dstg, isem).wait()

    def g_start(srcg, k, rows, gsem):
        pltpu.async_copy(y_hbm.at[srcg.at[k]], rows, gsem)

    def g_wait(srcg, k, rows, gsem):
        pltpu.make_async_copy(y_hbm.at[srcg.at[k]], rows, gsem).wait()

    def sc_do(dstg, k, rows, ssem):
        pltpu.async_copy(rows, agg_sh.at[dstg.at[k]], ssem, add=True)
        pltpu.make_async_copy(rows, agg_sh.at[dstg.at[k]], ssem).wait()

    i_start(1, srcgb, dstgb, isemb)
    g_start(srcga, 0, rows0, gsem0)
    g_start(srcga, 1, rows1, gsem1)

    rows_ = (rows0, rows1)
    gsem_ = (gsem0, gsem1)
    ssem_ = (ssem0, ssem1)

    def body(u, carry):
        more = u < NG // 2 - 1
        for cidx in range(2 * G):
            b = cidx % 2
            srcg, dstg = (srcga, dstga) if cidx < G else (srcgb, dstgb)
            k = cidx % G
            g_wait(srcg, k, rows_[b], gsem_[b])
            sc_do(dstg, k, rows_[b], ssem_[b])
            nxt = cidx + 2
            if nxt == G - 2:
                i_wait(2 * u + 1, srcgb, dstgb, isemb)
            if nxt < 2 * G:
                nsrc = srcga if nxt < G else srcgb
                g_start(nsrc, nxt % G, rows_[b], gsem_[b])
            else:

                @pl.when(more)
                def _():
                    if nxt == 2 * G:
                        i_wait(2 * u + 2, srcga, dstga, isema)
                    g_start(srcga, nxt % G, rows_[b], gsem_[b])

            if cidx == G - 1:

                @pl.when(more)
                def _():
                    i_start(2 * u + 2, srcga, dstga, isema)

            if cidx == 2 * G - 1:

                @pl.when(more)
                def _():
                    i_start(2 * u + 3, srcgb, dstgb, isemb)

        return carry

    lax.fori_loop(0, NG // 2, body, 0)
    plsc.subcore_barrier()

    wchunks = [(k * 128, 128) for k in range(4)] + [(512, NPTA - 512)]
    for off, ln in wchunks:
        pltpu.async_copy(agg_sh.at[pl.ds(s * NPTA + off, ln)],
                         out_hbm.at[c, pl.ds(s * NPTA + off, ln)], zsem)
    for off, ln in wchunks:
        pltpu.make_async_copy(agg_sh.at[pl.ds(s * NPTA + off, ln)],
                              out_hbm.at[c, pl.ds(s * NPTA + off, ln)],
                              zsem).wait()


@functools.lru_cache(maxsize=None)
def _sc_kernels():
    mesh = plsc.VectorSubcoreMesh(
        core_axis_name="c", subcore_axis_name="s",
        num_cores=NC, num_subcores=NS)
    deg_kernel = pl.kernel(
        _deg_body,
        out_type=jax.ShapeDtypeStruct((NC, NP), jnp.float32),
        mesh=mesh,
        scratch_types=[
            pltpu.VMEM((NG, G, CH), jnp.int32),
            pltpu.VMEM((128,), jnp.float32),
            pltpu.VMEM((NPT,), jnp.float32),
            pltpu.VMEM_SHARED((NP,), jnp.float32),
            pltpu.SemaphoreType.DMA,
        ],
    )
    agg_kernel = pl.kernel(
        _agg_body,
        out_type=jax.ShapeDtypeStruct((NC, NP, D), jnp.float32),
        mesh=mesh,
        scratch_types=[
            pltpu.VMEM((G, CH), jnp.int32),
            pltpu.VMEM((G, CH), jnp.int32),
            pltpu.VMEM((G, CH), jnp.int32),
            pltpu.VMEM((G, CH), jnp.int32),
            pltpu.VMEM((CH, D), jnp.float32),
            pltpu.VMEM((CH, D), jnp.float32),
            pltpu.VMEM((8, D), jnp.float32),
            pltpu.VMEM_SHARED((NPA, D), jnp.float32),
            pltpu.SemaphoreType.DMA,
            pltpu.SemaphoreType.DMA,
            pltpu.SemaphoreType.DMA,
            pltpu.SemaphoreType.DMA,
            pltpu.SemaphoreType.DMA,
            pltpu.SemaphoreType.DMA,
            pltpu.SemaphoreType.DMA,
        ],
    )
    return deg_kernel, agg_kernel


def _prep_body(degt_ref, x_ref, dinv_ref, y_ref):
    dinv = lax.rsqrt(degt_ref[:, 0:1] + degt_ref[:, 1:2] + 1.0)
    dinv_ref[...] = dinv
    y_ref[...] = x_ref[...] * dinv


def _prep(degt, x):
    blk = 1000
    grid = N // blk
    return pl.pallas_call(
        _prep_body,
        grid=(grid,),
        in_specs=[
            pl.BlockSpec((blk, NC), lambda i: (i, 0)),
            pl.BlockSpec((blk, D), lambda i: (i, 0)),
        ],
        out_specs=[
            pl.BlockSpec((blk, 1), lambda i: (i, 0)),
            pl.BlockSpec((blk, D), lambda i: (i, 0)),
        ],
        out_shape=[
            jax.ShapeDtypeStruct((N, 1), jnp.float32),
            jax.ShapeDtypeStruct((N, D), jnp.float32),
        ],
    )(degt, x)


def _final_body(p_ref, dinv_ref, x0_ref, x_ref, w_ref, o_ref):
    dv = dinv_ref[...]
    xv = x_ref[...]
    agg = (p_ref[0] + p_ref[1]) * dv + xv * (dv * dv)
    h = (1.0 - ALPHA) * agg + ALPHA * x0_ref[...]
    mm = jnp.dot(h, w_ref[...], preferred_element_type=jnp.float32)
    o_ref[...] = xv + jnp.maximum(mm, 0.0)


def _final(parts, dinv, x0, x, W1):
    blk = 1000
    grid = N // blk
    return pl.pallas_call(
        _final_body,
        grid=(grid,),
        in_specs=[
            pl.BlockSpec((NC, blk, D), lambda i: (0, i, 0)),
            pl.BlockSpec((blk, 1), lambda i: (i, 0)),
            pl.BlockSpec((blk, D), lambda i: (i, 0)),
            pl.BlockSpec((blk, D), lambda i: (i, 0)),
            pl.BlockSpec((D, D), lambda i: (0, 0)),
        ],
        out_specs=pl.BlockSpec((blk, D), lambda i: (i, 0)),
        out_shape=jax.ShapeDtypeStruct((N, D), jnp.float32),
    )(parts, dinv, x0, x, W1)


def kernel(x, x0, edge_index, W1):
    ei = edge_index.reshape(2, NW, NG, G, CH)
    deg_kernel, agg_kernel = _sc_kernels()
    degp = deg_kernel(ei)
    dinv, y = _prep(degp.T, x)
    parts = agg_kernel(y, ei)
    return _final(parts, dinv, x0, x, W1)
